# baseline (speedup 1.0000x reference)
import functools

import jax
import jax.numpy as jnp
from jax import lax
from jax.experimental import pallas as pl
from jax.experimental.pallas import tpu as pltpu
from jax.experimental.pallas import tpu_sc as plsc

_PERC = 0.25
_L = 16
_NSUB = 32
_NBIN = 256
_UNROLL = 8

_L1_GROUP_ORDER = list(range(7, -1, -1)) + list(range(15, 7, -1))


def _srl(v, n):
    return lax.shift_right_logical(v, jnp.full((_L,), n, jnp.int32))


def _zero_hist(hcnt):
    zi = jnp.zeros((_L,), jnp.int32)

    @plsc.parallel_loop(0, _NBIN, unroll=_UNROLL)
    def _(i):
        hcnt[pl.ds(i * _L, _L)] = zi


def _level_scan(hcnt, k_cur, lane_iota, group_order):
    best_bin = jnp.int32(-1)
    best_A = jnp.int32(0)
    carry = jnp.int32(0)
    for g in group_order:
        tot = jnp.zeros((_L,), jnp.int32)
        for l in range(_L):
            tot = tot + hcnt[pl.ds(l * _NBIN + g * _L, _L)]
        S = plsc.cumsum(tot)
        Tg = S[_L - 1]
        A = carry + Tg - S
        mask = (A < k_cur) & (A + tot >= k_cur)
        ids = g * _L + lane_iota
        best_bin = jnp.maximum(best_bin, jnp.max(jnp.where(mask, ids, -1)))
        best_A = jnp.maximum(best_A, jnp.max(jnp.where(mask, A, -1)))
        carry = carry + Tg
    return best_bin, best_A


def _sc_body(nrows_per_sub, nchunks, k, loss_hbm, out_hbm, data, hcnt, accv):
    int_min = jnp.int32(-(2**31))
    lane_iota = lax.iota(jnp.int32, _L)
    lane_base = lane_iota * _NBIN
    ones_i = jnp.ones((_L,), jnp.int32)
    desc = list(range(_NBIN // _L - 1, -1, -1))
    wid = lax.axis_index("s") * 2 + lax.axis_index("c")

    def row_body(r, acc):
        row = wid * nrows_per_sub + r
        pltpu.sync_copy(loss_hbm.at[row], data)

        _zero_hist(hcnt)

        @plsc.parallel_loop(0, nchunks, unroll=_UNROLL)
        def _(c):
            x = data[pl.ds(c * _L, _L)]
            bits = plsc.bitcast(x, jnp.int32)
            key = jnp.where(bits >= 0, bits, int_min - bits)
            data[pl.ds(c * _L, _L)] = plsc.bitcast(key, jnp.float32)
            plsc.addupdate_scatter(hcnt, [lane_base + _srl(key, 24)], ones_i)

        b1, A1 = _level_scan(hcnt, k, lane_iota, _L1_GROUP_ORDER)
        k2 = k - A1

        def masked_hist(shift, prefix):
            @plsc.parallel_loop(0, nchunks, unroll=_UNROLL)
            def _(c):
                key = plsc.bitcast(data[pl.ds(c * _L, _L)], jnp.int32)
                q = _srl(key, shift)
                m = _srl(q, 8) == prefix
                idx = lane_base + (q & 0xFF)
                plsc.addupdate_scatter(hcnt, [idx], ones_i, mask=m)

        _zero_hist(hcnt)
        masked_hist(16, b1)
        b2, A2 = _level_scan(hcnt, k2, lane_iota, desc)
        k3 = k2 - A2
        p16 = (b1 << 8) | b2

        _zero_hist(hcnt)
        masked_hist(8, p16)
        b3, A3 = _level_scan(hcnt, k3, lane_iota, desc)
        k4 = k3 - A3
        p24 = (p16 << 8) | b3

        _zero_hist(hcnt)
        masked_hist(0, p24)
        b4, A4 = _level_scan(hcnt, k4, lane_iota, desc)
        k5 = k4 - A4

        t_key = (p24 << 8) | b4
        zero_carry = (jnp.zeros((_L,), jnp.float32), jnp.zeros((_L,), jnp.int32))

        @plsc.parallel_loop(0, nchunks, unroll=_UNROLL, carry=zero_carry)
        def p5_acc(c, carry):
            sacc, cacc = carry
            key = plsc.bitcast(data[pl.ds(c * _L, _L)], jnp.int32)
            m = key > t_key
            xb = jnp.where(key >= 0, key, int_min - key)
            x = plsc.bitcast(xb, jnp.float32)
            return sacc + jnp.where(m, x, 0.0), cacc + m.astype(jnp.int32)

        sacc, cacc = p5_acc
        sum_gt = jnp.sum(sacc)
        cnt_gt = jnp.sum(cacc)

        t_bits = jnp.where(t_key >= 0, t_key, int_min - t_key)
        t_vec = plsc.bitcast(jnp.full((_L,), t_bits, jnp.int32), jnp.float32)
        t_f = t_vec[0]
        row_sum = sum_gt + (k - cnt_gt).astype(jnp.float32) * t_f
        return acc + row_sum

    acc = lax.fori_loop(0, nrows_per_sub, row_body, jnp.float32(0.0))
    accv[...] = jnp.where(lane_iota == 0, acc, 0.0)
    pltpu.sync_copy(accv, out_hbm.at[wid])


def kernel(loss):
    B = loss.shape[0]
    loss2 = loss.reshape(B, -1)
    P = loss2.shape[1]
    k = int(_PERC * P)
    nrows_per_sub = B // _NSUB
    nchunks = P // _L

    mesh = plsc.VectorSubcoreMesh(core_axis_name="c", subcore_axis_name="s")
    sc_call = pl.kernel(
        functools.partial(_sc_body, nrows_per_sub, nchunks, jnp.int32(k)),
        out_type=jax.ShapeDtypeStruct((_NSUB, _L), jnp.float32),
        mesh=mesh,
        compiler_params=pltpu.CompilerParams(needs_layout_passes=False),
        scratch_types=[
            pltpu.VMEM((P,), jnp.float32),
            pltpu.VMEM((_NBIN * _L,), jnp.int32),
            pltpu.VMEM((_L,), jnp.float32),
        ],
    )
    partial_sums = sc_call(loss2)
    return jnp.sum(partial_sums) / (B * k)


# speedup vs baseline: 1.0190x; 1.0190x over previous
import functools

import jax
import jax.numpy as jnp
from jax import lax
from jax.experimental import pallas as pl
from jax.experimental.pallas import tpu as pltpu
from jax.experimental.pallas import tpu_sc as plsc

_PERC = 0.25
_L = 16
_NSUB = 32
_NBIN = 256
_UNROLL = 8

_L1_GROUP_ORDER = list(range(7, -1, -1)) + list(range(15, 7, -1))


def _srl(v, n):
    return lax.shift_right_logical(v, jnp.full((_L,), n, jnp.int32))


def _zero_hist(hcnt):
    zi = jnp.zeros((_L,), jnp.int32)

    @plsc.parallel_loop(0, _NBIN, unroll=_UNROLL)
    def _(i):
        hcnt[pl.ds(i * _L, _L)] = zi


def _level_scan(hcnt, k_cur, lane_iota, group_order):
    best_bin = jnp.int32(-1)
    best_A = jnp.int32(0)
    carry = jnp.int32(0)
    for g in group_order:
        tot = jnp.zeros((_L,), jnp.int32)
        for l in range(_L):
            tot = tot + hcnt[pl.ds(l * _NBIN + g * _L, _L)]
        S = plsc.cumsum(tot)
        Tg = S[_L - 1]
        A = carry + Tg - S
        mask = (A < k_cur) & (A + tot >= k_cur)
        ids = g * _L + lane_iota
        best_bin = jnp.maximum(best_bin, jnp.max(jnp.where(mask, ids, -1)))
        best_A = jnp.maximum(best_A, jnp.max(jnp.where(mask, A, -1)))
        carry = carry + Tg
    return best_bin, best_A


def _sc_body(nrows_per_sub, nchunks, k, loss_hbm, out_hbm, data, hcnt, accv):
    int_min = jnp.int32(-(2**31))
    lane_iota = lax.iota(jnp.int32, _L)
    lane_base = lane_iota * _NBIN
    ones_i = jnp.ones((_L,), jnp.int32)
    desc = list(range(_NBIN // _L - 1, -1, -1))
    wid = lax.axis_index("s") * 2 + lax.axis_index("c")

    def row_body(r, acc):
        row = wid * nrows_per_sub + r
        pltpu.sync_copy(loss_hbm.at[row], data)

        _zero_hist(hcnt)

        @plsc.parallel_loop(0, nchunks, unroll=_UNROLL)
        def _(c):
            x = data[pl.ds(c * _L, _L)]
            bits = plsc.bitcast(x, jnp.int32)
            key = jnp.where(bits >= 0, bits, int_min - bits)
            data[pl.ds(c * _L, _L)] = plsc.bitcast(key, jnp.float32)
            plsc.addupdate_scatter(hcnt, [lane_base + _srl(key, 24)], ones_i)

        b1, A1 = _level_scan(hcnt, k, lane_iota, _L1_GROUP_ORDER)
        k2 = k - A1

        def masked_hist(shift, prefix):
            @plsc.parallel_loop(0, nchunks, unroll=_UNROLL)
            def _(c):
                key = plsc.bitcast(data[pl.ds(c * _L, _L)], jnp.int32)
                q = _srl(key, shift)
                m = _srl(q, 8) == prefix
                idx = lane_base + (q & 0xFF)
                plsc.addupdate_scatter(hcnt, [idx], ones_i, mask=m)

        _zero_hist(hcnt)
        masked_hist(16, b1)
        b2, A2 = _level_scan(hcnt, k2, lane_iota, desc)
        k3 = k2 - A2
        p16 = (b1 << 8) | b2

        _zero_hist(hcnt)
        masked_hist(8, p16)
        b3, A3 = _level_scan(hcnt, k3, lane_iota, desc)
        k4 = k3 - A3
        p24 = (p16 << 8) | b3

        _zero_hist(hcnt)
        masked_hist(0, p24)
        b4, A4 = _level_scan(hcnt, k4, lane_iota, desc)
        k5 = k4 - A4

        t_key = (p24 << 8) | b4

        @plsc.parallel_loop(0, nchunks, unroll=_UNROLL,
                            carry=jnp.zeros((_L,), jnp.float32))
        def p5_acc(c, sacc):
            key = plsc.bitcast(data[pl.ds(c * _L, _L)], jnp.int32)
            m = key > t_key
            xb = jnp.where(key >= 0, key, int_min - key)
            x = plsc.bitcast(xb, jnp.float32)
            return sacc + jnp.where(m, x, 0.0)

        sum_gt = jnp.sum(p5_acc)

        t_bits = jnp.where(t_key >= 0, t_key, int_min - t_key)
        t_vec = plsc.bitcast(jnp.full((_L,), t_bits, jnp.int32), jnp.float32)
        t_f = t_vec[0]
        row_sum = sum_gt + k5.astype(jnp.float32) * t_f
        return acc + row_sum

    acc = lax.fori_loop(0, nrows_per_sub, row_body, jnp.float32(0.0))
    accv[...] = jnp.where(lane_iota == 0, acc, 0.0)
    pltpu.sync_copy(accv, out_hbm.at[wid])


def kernel(loss):
    B = loss.shape[0]
    loss2 = loss.reshape(B, -1)
    P = loss2.shape[1]
    k = int(_PERC * P)
    nrows_per_sub = B // _NSUB
    nchunks = P // _L

    mesh = plsc.VectorSubcoreMesh(core_axis_name="c", subcore_axis_name="s")
    sc_call = pl.kernel(
        functools.partial(_sc_body, nrows_per_sub, nchunks, jnp.int32(k)),
        out_type=jax.ShapeDtypeStruct((_NSUB, _L), jnp.float32),
        mesh=mesh,
        compiler_params=pltpu.CompilerParams(needs_layout_passes=False),
        scratch_types=[
            pltpu.VMEM((P,), jnp.float32),
            pltpu.VMEM((_NBIN * _L,), jnp.int32),
            pltpu.VMEM((_L,), jnp.float32),
        ],
    )
    partial_sums = sc_call(loss2)
    return jnp.sum(partial_sums) / (B * k)
